# Initial kernel scaffold; baseline (speedup 1.0000x reference)
import jax
import jax.numpy as jnp
from jax.experimental import pallas as pl


def kernel(doc, query, query_idf, emb, ffw_W, ffw_b, gates_W, out_W, out_b):
    raise NotImplementedError("write your pallas kernel here")


# speedup vs baseline: 4.3991x; 4.3991x over previous
import functools

import jax
import jax.numpy as jnp
from jax import lax
from jax.experimental import pallas as pl
from jax.experimental.pallas import tpu as pltpu
from jax.experimental.pallas import tpu_sc as plsc

_INT_MIN = -2147483648


def _gather_rows(emb, q_idx, d_idx):
    E = emb.shape[1]
    Nq = q_idx.shape[1]
    Nd = d_idx.shape[1]
    W = 128
    mesh = plsc.VectorSubcoreMesh(core_axis_name="core", subcore_axis_name="subcore")

    @functools.partial(
        pl.kernel,
        out_type=(
            jax.ShapeDtypeStruct((Nq, E), emb.dtype),
            jax.ShapeDtypeStruct((Nd, E), emb.dtype),
        ),
        mesh=mesh,
        compiler_params=pltpu.CompilerParams(use_tc_tiling_on_sc=False),
    )
    def gather_kernel(emb_hbm, qi_hbm, di_hbm, qo_hbm, do_hbm):
        def body(i_vmem, o_vmem):
            pltpu.sync_copy(emb_hbm.at[i_vmem.at[0]], o_vmem)

        pltpu.emit_pipeline(
            body,
            grid=(Nq // W,),
            in_specs=[pl.BlockSpec((1, W), lambda i: (0, i))],
            out_specs=[pl.BlockSpec((W, E), lambda i: (i, 0))],
            core_axis_name=("core", "subcore"),
            dimension_semantics=(pltpu.PARALLEL,),
        )(qi_hbm, qo_hbm)
        pltpu.emit_pipeline(
            body,
            grid=(Nd // W,),
            in_specs=[pl.BlockSpec((1, W), lambda i: (0, i))],
            out_specs=[pl.BlockSpec((W, E), lambda i: (i, 0))],
            core_axis_name=("core", "subcore"),
            dimension_semantics=(pltpu.PARALLEL,),
        )(di_hbm, do_hbm)

    return gather_kernel(emb, q_idx, d_idx)


def _scores(qe_all, de_all, query, query_idf, ffw_W, ffw_b, gates_W, out_W,
            out_b, G=8, interpret=False):
    B, TQ = query.shape
    TD = de_all.shape[0] // B
    E = qe_all.shape[1]
    K = ffw_W.shape[1]

    def body(q_ref, idf_ref, qe_ref, de_ref, fw_ref, fb_ref, gw_ref, ow_ref,
             ob_ref, out_ref):
        qe = qe_ref[...]
        de = de_ref[...]
        qn = qe / (jnp.sqrt(jnp.sum(qe * qe, axis=1, keepdims=True)) + 1e-8)
        dn = de / (jnp.sqrt(jnp.sum(de * de, axis=1, keepdims=True)) + 1e-8)
        cos_rows = []
        for j in range(G):
            qj = qn[j * TQ:(j + 1) * TQ, :]
            dj = dn[j * TD:(j + 1) * TD, :]
            cos_rows.append(
                lax.dot_general(qj, dj, (((1,), (1,)), ((), ())),
                                preferred_element_type=jnp.float32))
        cos = jnp.concatenate(cos_rows, axis=0)

        bits = lax.bitcast_convert_type(cos, jnp.int32)
        key = jnp.where(bits >= 0, bits, bits ^ jnp.int32(0x7FFFFFFF))
        col = lax.broadcasted_iota(jnp.int32, (G * TQ, TD), 1)
        key = (key & jnp.int32(-256)) | col
        acc = jnp.zeros((G * TQ, 1), jnp.float32)
        for k in range(K):
            mk = jnp.max(key, axis=1, keepdims=True)
            vb = mk & jnp.int32(-256)
            vb = jnp.where(vb >= 0, vb, vb ^ jnp.int32(0x7FFFFFFF))
            val = lax.bitcast_convert_type(vb, jnp.float32)
            acc = acc + fw_ref[0, k] * val
            key = jnp.where(key == mk, jnp.int32(_INT_MIN), key)
        f = jnp.tanh(acc + fb_ref[0])

        q = q_ref[...]
        idf = idf_ref[...]
        logits = idf * gw_ref[0, 0] + jnp.where(
            q == 0, jnp.float32(-1e7), jnp.float32(0.0))
        mx = jnp.max(logits, axis=1, keepdims=True)
        ex = jnp.exp(logits - mx)
        p = ex / jnp.sum(ex, axis=1, keepdims=True)

        outs = []
        for j in range(G):
            pj = p[j:j + 1, :]
            fj = f[j * TQ:(j + 1) * TQ, :]
            outs.append(
                lax.dot_general(pj, fj, (((1,), (0,)), ((), ())),
                                preferred_element_type=jnp.float32))
        sc = jnp.concatenate(outs, axis=0)
        out_ref[...] = sc * ow_ref[0, 0] + ob_ref[0]

    return pl.pallas_call(
        body,
        grid=(B // G,),
        in_specs=[
            pl.BlockSpec((G, TQ), lambda i: (i, 0)),
            pl.BlockSpec((G, TQ), lambda i: (i, 0)),
            pl.BlockSpec((G * TQ, E), lambda i: (i, 0)),
            pl.BlockSpec((G * TD, E), lambda i: (i, 0)),
            pl.BlockSpec(memory_space=pltpu.SMEM),
            pl.BlockSpec(memory_space=pltpu.SMEM),
            pl.BlockSpec(memory_space=pltpu.SMEM),
            pl.BlockSpec(memory_space=pltpu.SMEM),
            pl.BlockSpec(memory_space=pltpu.SMEM),
        ],
        out_specs=pl.BlockSpec((G, 1), lambda i: (i, 0)),
        out_shape=jax.ShapeDtypeStruct((B, 1), jnp.float32),
        interpret=interpret,
    )(query, query_idf, qe_all, de_all, ffw_W, ffw_b, gates_W, out_W, out_b)


def kernel(doc, query, query_idf, emb, ffw_W, ffw_b, gates_W, out_W, out_b):
    B, TQ = query.shape
    TD = doc.shape[1]
    q_idx = query.reshape(1, B * TQ)
    d_idx = doc.reshape(1, B * TD)
    qe_all, de_all = _gather_rows(emb, q_idx, d_idx)
    return _scores(qe_all, de_all, query, query_idf, ffw_W, ffw_b, gates_W,
                   out_W, out_b)


# speedup vs baseline: 5.9271x; 1.3473x over previous
import functools

import jax
import jax.numpy as jnp
from jax import lax
from jax.experimental import pallas as pl
from jax.experimental.pallas import tpu as pltpu
from jax.experimental.pallas import tpu_sc as plsc

_INT_MIN = -2147483648


def _gather_rows(emb, q_idx, d_idx):
    E = emb.shape[1]
    Nq = q_idx.shape[1]
    Nd = d_idx.shape[1]
    W = 128
    mesh = plsc.VectorSubcoreMesh(core_axis_name="core", subcore_axis_name="subcore")

    @functools.partial(
        pl.kernel,
        out_type=(
            jax.ShapeDtypeStruct((Nq, E), emb.dtype),
            jax.ShapeDtypeStruct((Nd, E), emb.dtype),
        ),
        mesh=mesh,
        compiler_params=pltpu.CompilerParams(use_tc_tiling_on_sc=False),
    )
    def gather_kernel(emb_hbm, qi_hbm, di_hbm, qo_hbm, do_hbm):
        def body(i_vmem, o_vmem):
            pltpu.sync_copy(emb_hbm.at[i_vmem.at[0]], o_vmem)

        pltpu.emit_pipeline(
            body,
            grid=(Nq // W,),
            in_specs=[pl.BlockSpec((1, W), lambda i: (0, i))],
            out_specs=[pl.BlockSpec((W, E), lambda i: (i, 0))],
            core_axis_name=("core", "subcore"),
            dimension_semantics=(pltpu.PARALLEL,),
        )(qi_hbm, qo_hbm)
        pltpu.emit_pipeline(
            body,
            grid=(Nd // W,),
            in_specs=[pl.BlockSpec((1, W), lambda i: (0, i))],
            out_specs=[pl.BlockSpec((W, E), lambda i: (i, 0))],
            core_axis_name=("core", "subcore"),
            dimension_semantics=(pltpu.PARALLEL,),
        )(di_hbm, do_hbm)

    return gather_kernel(emb, q_idx, d_idx)


def _scores(qe_all, de_all, query, query_idf, ffw_W, ffw_b, gates_W, out_W,
            out_b, G=8, interpret=False):
    B, TQ = query.shape
    TD = de_all.shape[0] // B
    E = qe_all.shape[1]
    K = ffw_W.shape[1]

    def body(q_ref, idf_ref, qe_ref, de_ref, fw_ref, fb_ref, gw_ref, ow_ref,
             ob_ref, out_ref):
        qe = qe_ref[...]
        de = de_ref[...]
        qn = qe * (1.0 / (jnp.sqrt(jnp.sum(qe * qe, axis=1, keepdims=True)) + 1e-8))
        dn = de * (1.0 / (jnp.sqrt(jnp.sum(de * de, axis=1, keepdims=True)) + 1e-8))
        cos_rows = []
        for j in range(G):
            qj = qn[j * TQ:(j + 1) * TQ, :]
            dj = dn[j * TD:(j + 1) * TD, :]
            cos_rows.append(
                lax.dot_general(qj, dj, (((1,), (1,)), ((), ())),
                                preferred_element_type=jnp.float32))
        cos = jnp.concatenate(cos_rows, axis=0)

        bits = lax.bitcast_convert_type(cos, jnp.int32)
        skey = jnp.where(bits >= 0, bits, bits ^ jnp.int32(0x7FFFFFFF))
        col = lax.broadcasted_iota(jnp.int32, (G * TQ, TD), 1)
        skey = (skey & jnp.int32(-256)) | col
        skey = jnp.where(skey >= 0, skey, skey ^ jnp.int32(0x7FFFFFFF))
        key = lax.bitcast_convert_type(skey, jnp.float32)
        acc = jnp.zeros((G * TQ, 1), jnp.float32)
        neg_inf = jnp.float32(-jnp.inf)
        for k in range(K):
            mk = jnp.max(key, axis=1, keepdims=True)
            acc = acc + fw_ref[0, k] * mk
            key = jnp.where(key == mk, neg_inf, key)
        f = jnp.tanh(acc + fb_ref[0])

        q = q_ref[...]
        idf = idf_ref[...]
        logits = idf * gw_ref[0, 0] + jnp.where(
            q == 0, jnp.float32(-1e7), jnp.float32(0.0))
        mx = jnp.max(logits, axis=1, keepdims=True)
        ex = jnp.exp(logits - mx)
        p = ex / jnp.sum(ex, axis=1, keepdims=True)

        outs = []
        for j in range(G):
            pj = p[j:j + 1, :]
            fj = f[j * TQ:(j + 1) * TQ, :]
            outs.append(
                lax.dot_general(pj, fj, (((1,), (0,)), ((), ())),
                                preferred_element_type=jnp.float32))
        sc = jnp.concatenate(outs, axis=0)
        out_ref[...] = sc * ow_ref[0, 0] + ob_ref[0]

    return pl.pallas_call(
        body,
        grid=(B // G,),
        in_specs=[
            pl.BlockSpec((G, TQ), lambda i: (i, 0)),
            pl.BlockSpec((G, TQ), lambda i: (i, 0)),
            pl.BlockSpec((G * TQ, E), lambda i: (i, 0)),
            pl.BlockSpec((G * TD, E), lambda i: (i, 0)),
            pl.BlockSpec(memory_space=pltpu.SMEM),
            pl.BlockSpec(memory_space=pltpu.SMEM),
            pl.BlockSpec(memory_space=pltpu.SMEM),
            pl.BlockSpec(memory_space=pltpu.SMEM),
            pl.BlockSpec(memory_space=pltpu.SMEM),
        ],
        out_specs=pl.BlockSpec((G, 1), lambda i: (i, 0)),
        out_shape=jax.ShapeDtypeStruct((B, 1), jnp.float32),
        interpret=interpret,
    )(query, query_idf, qe_all, de_all, ffw_W, ffw_b, gates_W, out_W, out_b)


def kernel(doc, query, query_idf, emb, ffw_W, ffw_b, gates_W, out_W, out_b):
    B, TQ = query.shape
    TD = doc.shape[1]
    q_idx = query.reshape(1, B * TQ)
    d_idx = doc.reshape(1, B * TD)
    qe_all, de_all = _gather_rows(emb, q_idx, d_idx)
    return _scores(qe_all, de_all, query, query_idf, ffw_W, ffw_b, gates_W,
                   out_W, out_b)


# speedup vs baseline: 6.1025x; 1.0296x over previous
import functools

import jax
import jax.numpy as jnp
from jax import lax
from jax.experimental import pallas as pl
from jax.experimental.pallas import tpu as pltpu
from jax.experimental.pallas import tpu_sc as plsc

_INT_MIN = -2147483648


def _gather_rows(emb, q_idx, d_idx):
    E = emb.shape[1]
    Nq = q_idx.shape[1]
    Nd = d_idx.shape[1]
    W = 128
    mesh = plsc.VectorSubcoreMesh(core_axis_name="core", subcore_axis_name="subcore")

    @functools.partial(
        pl.kernel,
        out_type=(
            jax.ShapeDtypeStruct((Nq, E), emb.dtype),
            jax.ShapeDtypeStruct((Nd, E), emb.dtype),
        ),
        mesh=mesh,
        compiler_params=pltpu.CompilerParams(use_tc_tiling_on_sc=False),
    )
    def gather_kernel(emb_hbm, qi_hbm, di_hbm, qo_hbm, do_hbm):
        def body(i_vmem, o_vmem):
            pltpu.sync_copy(emb_hbm.at[i_vmem.at[0]], o_vmem)

        pltpu.emit_pipeline(
            body,
            grid=(Nq // W,),
            in_specs=[pl.BlockSpec((1, W), lambda i: (0, i))],
            out_specs=[pl.BlockSpec((W, E), lambda i: (i, 0))],
            core_axis_name=("core", "subcore"),
            dimension_semantics=(pltpu.PARALLEL,),
        )(qi_hbm, qo_hbm)
        pltpu.emit_pipeline(
            body,
            grid=(Nd // W,),
            in_specs=[pl.BlockSpec((1, W), lambda i: (0, i))],
            out_specs=[pl.BlockSpec((W, E), lambda i: (i, 0))],
            core_axis_name=("core", "subcore"),
            dimension_semantics=(pltpu.PARALLEL,),
        )(di_hbm, do_hbm)

    return gather_kernel(emb, q_idx, d_idx)


def _scores(qe_all, de_all, query, query_idf, ffw_W, ffw_b, gates_W, out_W,
            out_b, G=8, interpret=False):
    B, TQ = query.shape
    TD = de_all.shape[0] // B
    E = qe_all.shape[1]
    K = ffw_W.shape[1]

    def body(q_ref, idf_ref, qe_ref, de_ref, fw_ref, fb_ref, gw_ref, ow_ref,
             ob_ref, out_ref):
        qe = qe_ref[...]
        de = de_ref[...]
        qn = qe * (1.0 / (jnp.sqrt(jnp.sum(qe * qe, axis=1, keepdims=True)) + 1e-8))
        dn = de * (1.0 / (jnp.sqrt(jnp.sum(de * de, axis=1, keepdims=True)) + 1e-8))
        cos_rows = []
        for j in range(G):
            qj = qn[j * TQ:(j + 1) * TQ, :]
            dj = dn[j * TD:(j + 1) * TD, :]
            cos_rows.append(
                lax.dot_general(qj, dj, (((1,), (1,)), ((), ())),
                                preferred_element_type=jnp.float32))
        cos = jnp.concatenate(cos_rows, axis=0)

        bits = lax.bitcast_convert_type(cos, jnp.int32)
        skey = jnp.where(bits >= 0, bits, bits ^ jnp.int32(0x7FFFFFFF))
        col = lax.broadcasted_iota(jnp.int32, (G * TQ, TD), 1)
        skey = (skey & jnp.int32(-256)) | col
        skey = jnp.where(skey >= 0, skey, skey ^ jnp.int32(0x7FFFFFFF))
        key = lax.bitcast_convert_type(skey, jnp.float32)

        neg_inf = jnp.float32(-jnp.inf)
        kA = key[:, :128]
        kB = jnp.concatenate(
            [key[:, 128:],
             jnp.full((G * TQ, 256 - TD), neg_inf, jnp.float32)], axis=1)
        fm = jnp.maximum(kA, kB)
        sm = jnp.minimum(kA, kB)
        tops = []
        for _ in range(K):
            mk = jnp.max(fm, axis=1, keepdims=True)
            tops.append(mk)
            c = fm == mk
            fm = jnp.where(c, sm, fm)
            sm = jnp.where(c, neg_inf, sm)
        topm = jnp.concatenate(tops, axis=1)

        tb = lax.bitcast_convert_type(topm, jnp.int32)
        tsk = jnp.where(tb >= 0, tb, tb ^ jnp.int32(0x7FFFFFFF))
        tsk = (tsk & jnp.int32(-256)) | jnp.int32(128)
        tsk = jnp.where(tsk >= 0, tsk, tsk ^ jnp.int32(0x7FFFFFFF))
        vals = lax.bitcast_convert_type(tsk, jnp.float32)
        acc = jnp.sum(vals * fw_ref[...], axis=1, keepdims=True)
        f = jnp.tanh(acc + fb_ref[0])

        q = q_ref[...]
        idf = idf_ref[...]
        logits = idf * gw_ref[0, 0] + jnp.where(
            q == 0, jnp.float32(-1e7), jnp.float32(0.0))
        mx = jnp.max(logits, axis=1, keepdims=True)
        ex = jnp.exp(logits - mx)
        p = ex / jnp.sum(ex, axis=1, keepdims=True)

        outs = []
        for j in range(G):
            pj = p[j:j + 1, :]
            fj = f[j * TQ:(j + 1) * TQ, :]
            outs.append(
                lax.dot_general(pj, fj, (((1,), (0,)), ((), ())),
                                preferred_element_type=jnp.float32))
        sc = jnp.concatenate(outs, axis=0)
        out_ref[...] = sc * ow_ref[0, 0] + ob_ref[0]

    return pl.pallas_call(
        body,
        grid=(B // G,),
        in_specs=[
            pl.BlockSpec((G, TQ), lambda i: (i, 0)),
            pl.BlockSpec((G, TQ), lambda i: (i, 0)),
            pl.BlockSpec((G * TQ, E), lambda i: (i, 0)),
            pl.BlockSpec((G * TD, E), lambda i: (i, 0)),
            pl.BlockSpec((1, K), lambda i: (0, 0)),
            pl.BlockSpec(memory_space=pltpu.SMEM),
            pl.BlockSpec(memory_space=pltpu.SMEM),
            pl.BlockSpec(memory_space=pltpu.SMEM),
            pl.BlockSpec(memory_space=pltpu.SMEM),
        ],
        out_specs=pl.BlockSpec((G, 1), lambda i: (i, 0)),
        out_shape=jax.ShapeDtypeStruct((B, 1), jnp.float32),
        interpret=interpret,
    )(query, query_idf, qe_all, de_all, ffw_W, ffw_b, gates_W, out_W, out_b)


def kernel(doc, query, query_idf, emb, ffw_W, ffw_b, gates_W, out_W, out_b):
    B, TQ = query.shape
    TD = doc.shape[1]
    S = 2
    Bs = B // S
    gathered = []
    for h in range(S):
        qs = query[h * Bs:(h + 1) * Bs]
        ds = doc[h * Bs:(h + 1) * Bs]
        gathered.append(_gather_rows(emb, qs.reshape(1, Bs * TQ),
                                     ds.reshape(1, Bs * TD)))
    outs = []
    for h in range(S):
        qe_all, de_all = gathered[h]
        outs.append(
            _scores(qe_all, de_all, query[h * Bs:(h + 1) * Bs],
                    query_idf[h * Bs:(h + 1) * Bs], ffw_W, ffw_b, gates_W,
                    out_W, out_b))
    return jnp.concatenate(outs, axis=0)


# speedup vs baseline: 8.8689x; 1.4533x over previous
import functools

import jax
import jax.numpy as jnp
from jax import lax
from jax.experimental import pallas as pl
from jax.experimental.pallas import tpu as pltpu
from jax.experimental.pallas import tpu_sc as plsc

_INT_MIN = -2147483648


def _gather_rows(emb, q_idx, d_idx):
    E = emb.shape[1]
    Nq = q_idx.shape[1]
    Nd = d_idx.shape[1]
    W = 128
    mesh = plsc.VectorSubcoreMesh(core_axis_name="core", subcore_axis_name="subcore")

    @functools.partial(
        pl.kernel,
        out_type=(
            jax.ShapeDtypeStruct((Nq, E), emb.dtype),
            jax.ShapeDtypeStruct((Nd, E), emb.dtype),
        ),
        mesh=mesh,
        compiler_params=pltpu.CompilerParams(use_tc_tiling_on_sc=False),
    )
    def gather_kernel(emb_hbm, qi_hbm, di_hbm, qo_hbm, do_hbm):
        def body(i_vmem, o_vmem):
            pltpu.sync_copy(emb_hbm.at[i_vmem.at[0]], o_vmem)

        pltpu.emit_pipeline(
            body,
            grid=(Nq // W,),
            in_specs=[pl.BlockSpec((1, W), lambda i: (0, i))],
            out_specs=[pl.BlockSpec((W, E), lambda i: (i, 0))],
            core_axis_name=("core", "subcore"),
            dimension_semantics=(pltpu.PARALLEL,),
        )(qi_hbm, qo_hbm)
        pltpu.emit_pipeline(
            body,
            grid=(Nd // W,),
            in_specs=[pl.BlockSpec((1, W), lambda i: (0, i))],
            out_specs=[pl.BlockSpec((W, E), lambda i: (i, 0))],
            core_axis_name=("core", "subcore"),
            dimension_semantics=(pltpu.PARALLEL,),
        )(di_hbm, do_hbm)

    return gather_kernel(emb, q_idx, d_idx)


def _scores(qe_all, de_all, query, query_idf, ffw_W, ffw_b, gates_W, out_W,
            out_b, G=32, interpret=False):
    B, TQ = query.shape
    TD = de_all.shape[0] // B
    E = qe_all.shape[1]
    K = ffw_W.shape[1]

    def body(q_ref, idf_ref, qe_ref, de_ref, fw_ref, fb_ref, gw_ref, ow_ref,
             ob_ref, out_ref):
        qe = qe_ref[...]
        de = de_ref[...]
        qn = qe * (1.0 / (jnp.sqrt(jnp.sum(qe * qe, axis=1, keepdims=True)) + 1e-8))
        dn = de * (1.0 / (jnp.sqrt(jnp.sum(de * de, axis=1, keepdims=True)) + 1e-8))
        cos_rows = []
        for j in range(G):
            qj = qn[j * TQ:(j + 1) * TQ, :]
            dj = dn[j * TD:(j + 1) * TD, :]
            cos_rows.append(
                lax.dot_general(qj, dj, (((1,), (1,)), ((), ())),
                                preferred_element_type=jnp.float32))
        cos = jnp.concatenate(cos_rows, axis=0)

        bits = lax.bitcast_convert_type(cos, jnp.int32)
        skey = jnp.where(bits >= 0, bits, bits ^ jnp.int32(0x7FFFFFFF))
        col = lax.broadcasted_iota(jnp.int32, (G * TQ, TD), 1)
        skey = (skey & jnp.int32(-256)) | col
        skey = jnp.where(skey >= 0, skey, skey ^ jnp.int32(0x7FFFFFFF))
        key = lax.bitcast_convert_type(skey, jnp.float32)

        neg_inf = jnp.float32(-jnp.inf)
        kA = key[:, :128]
        kB = jnp.concatenate(
            [key[:, 128:],
             jnp.full((G * TQ, 256 - TD), neg_inf, jnp.float32)], axis=1)
        fm = jnp.maximum(kA, kB)
        sm = jnp.minimum(kA, kB)
        tops = []
        for _ in range(K):
            mk = jnp.max(fm, axis=1, keepdims=True)
            tops.append(mk)
            c = fm == mk
            fm = jnp.where(c, sm, fm)
            sm = jnp.where(c, neg_inf, sm)
        topm = jnp.concatenate(tops, axis=1)

        tb = lax.bitcast_convert_type(topm, jnp.int32)
        tsk = jnp.where(tb >= 0, tb, tb ^ jnp.int32(0x7FFFFFFF))
        tsk = (tsk & jnp.int32(-256)) | jnp.int32(128)
        tsk = jnp.where(tsk >= 0, tsk, tsk ^ jnp.int32(0x7FFFFFFF))
        vals = lax.bitcast_convert_type(tsk, jnp.float32)
        acc = jnp.sum(vals * fw_ref[...], axis=1, keepdims=True)
        f = jnp.tanh(acc + fb_ref[0])

        q = q_ref[...]
        idf = idf_ref[...]
        logits = idf * gw_ref[0, 0] + jnp.where(
            q == 0, jnp.float32(-1e7), jnp.float32(0.0))
        mx = jnp.max(logits, axis=1, keepdims=True)
        ex = jnp.exp(logits - mx)
        p = ex / jnp.sum(ex, axis=1, keepdims=True)

        outs = []
        for j in range(G):
            pj = p[j:j + 1, :]
            fj = f[j * TQ:(j + 1) * TQ, :]
            outs.append(
                lax.dot_general(pj, fj, (((1,), (0,)), ((), ())),
                                preferred_element_type=jnp.float32))
        sc = jnp.concatenate(outs, axis=0)
        out_ref[...] = sc * ow_ref[0, 0] + ob_ref[0]

    return pl.pallas_call(
        body,
        grid=(B // G,),
        in_specs=[
            pl.BlockSpec((G, TQ), lambda i: (i, 0)),
            pl.BlockSpec((G, TQ), lambda i: (i, 0)),
            pl.BlockSpec((G * TQ, E), lambda i: (i, 0)),
            pl.BlockSpec((G * TD, E), lambda i: (i, 0)),
            pl.BlockSpec((1, K), lambda i: (0, 0)),
            pl.BlockSpec(memory_space=pltpu.SMEM),
            pl.BlockSpec(memory_space=pltpu.SMEM),
            pl.BlockSpec(memory_space=pltpu.SMEM),
            pl.BlockSpec(memory_space=pltpu.SMEM),
        ],
        out_specs=pl.BlockSpec((G, 1), lambda i: (i, 0)),
        out_shape=jax.ShapeDtypeStruct((B, 1), jnp.float32),
        interpret=interpret,
    )(query, query_idf, qe_all, de_all, ffw_W, ffw_b, gates_W, out_W, out_b)


def kernel(doc, query, query_idf, emb, ffw_W, ffw_b, gates_W, out_W, out_b):
    B, TQ = query.shape
    TD = doc.shape[1]
    S = 2
    Bs = B // S
    gathered = []
    for h in range(S):
        qs = query[h * Bs:(h + 1) * Bs]
        ds = doc[h * Bs:(h + 1) * Bs]
        gathered.append(_gather_rows(emb, qs.reshape(1, Bs * TQ),
                                     ds.reshape(1, Bs * TD)))
    outs = []
    for h in range(S):
        qe_all, de_all = gathered[h]
        outs.append(
            _scores(qe_all, de_all, query[h * Bs:(h + 1) * Bs],
                    query_idf[h * Bs:(h + 1) * Bs], ffw_W, ffw_b, gates_W,
                    out_W, out_b))
    return jnp.concatenate(outs, axis=0)
